# baseline (speedup 1.0000x reference)
import functools

import jax
import jax.numpy as jnp
from jax import lax
from jax.experimental import pallas as pl
from jax.experimental.pallas import tpu as pltpu
from jax.experimental.pallas import tpu_sc as plsc

_N = 10000
_E = 320000
_D = 128
_G = 64
_OUT = 8

_NC = 2
_NS = 16
_NW = _NC * _NS

_EPT = _E // _NW
_CH = 80
_NCH = _EPT // _CH
_K = 2
_NG = (_NCH - 1) // _K
_RPT = _N // _NS
_RCH = 125
_NRC = _RPT // _RCH
_ZCH = 25
_NZC = _RPT // _ZCH


def _segsum_body(x_hbm, src_hbm, dst_hbm, out_hbm,
                 src_v, dst_v, r0, r1, zbuf, acc, s0, s1):
    rows = [r0, r1]
    sems = [s0, s1]
    c = lax.axis_index("c")
    s = lax.axis_index("s")
    wid = c * _NS + s

    pltpu.sync_copy(src_hbm.at[wid], src_v)
    pltpu.sync_copy(dst_hbm.at[wid], dst_v)
    for b in range(_K):
        pltpu.async_copy(x_hbm.at[src_v.at[b]], rows[b], sems[b])

    zero16 = jnp.zeros((16,), jnp.float32)

    def _z(i, carry):
        for k in range(_D // 16):
            zbuf[i, pl.ds(k * 16, 16)] = zero16
        return carry

    lax.fori_loop(0, _ZCH, _z, 0)

    def _zc(r, carry):
        pltpu.sync_copy(zbuf, acc.at[pl.ds(s * _RPT + r * _ZCH, _ZCH)])
        return carry

    lax.fori_loop(0, _NZC, _zc, 0)
    plsc.subcore_barrier()

    def _grp(g, carry):
        for b in range(_K):
            j = g * _K + b
            pltpu.make_async_copy(x_hbm.at[src_v.at[j]], rows[b],
                                  sems[b]).wait()
            pltpu.sync_copy(rows[b], acc.at[dst_v.at[j]], add=True)
            if b == 0:
                pltpu.async_copy(x_hbm.at[src_v.at[j + _K]], rows[b],
                                 sems[b])
            else:
                @pl.when(g < _NG - 1)
                def _():
                    pltpu.async_copy(x_hbm.at[src_v.at[j + _K]], rows[b],
                                     sems[b])
        return carry

    lax.fori_loop(0, _NG, _grp, 0)
    jt = _NCH - 1
    pltpu.make_async_copy(x_hbm.at[src_v.at[jt]], rows[0], sems[0]).wait()
    pltpu.sync_copy(rows[0], acc.at[dst_v.at[jt]], add=True)
    plsc.subcore_barrier()

    for r in range(_NRC):
        off = s * _RPT + r * _RCH
        pltpu.sync_copy(acc.at[pl.ds(off, _RCH)],
                        out_hbm.at[pl.ds(c * _N + off, _RCH)])


_segsum = functools.partial(
    pl.kernel,
    out_type=jax.ShapeDtypeStruct((2 * _N, _D), jnp.float32),
    mesh=plsc.VectorSubcoreMesh(core_axis_name="c", subcore_axis_name="s"),
    scratch_types=(
        [pltpu.VMEM((_NCH, _CH), jnp.int32),
         pltpu.VMEM((_NCH, _CH), jnp.int32)]
        + [pltpu.VMEM((_CH, _D), jnp.float32) for _ in range(_K)]
        + [pltpu.VMEM((_ZCH, _D), jnp.float32),
           pltpu.VMEM_SHARED((_N, _D), jnp.float32)]
        + [pltpu.SemaphoreType.DMA for _ in range(_K)]
    ),
    compiler_params=pltpu.CompilerParams(use_tc_tiling_on_sc=False),
)(_segsum_body)


def _bn(v, g, be):
    m = jnp.mean(v, axis=0)
    var = jnp.mean((v - m) * (v - m), axis=0)
    return (v - m) / jnp.sqrt(var + 1e-5) * g + be


def _layer_body(x_ref, p_ref, Wa_ref, ba_ref, Wb_ref, bb_ref, g_ref, be_ref,
                o_ref):
    h = x_ref[...] + p_ref[0] + p_ref[1]
    t = jnp.maximum(
        jnp.dot(h, Wa_ref[...], preferred_element_type=jnp.float32)
        + ba_ref[...], 0.0)
    u = (jnp.dot(t, Wb_ref[...], preferred_element_type=jnp.float32)
         + bb_ref[...])
    v = jnp.maximum(u, 0.0)
    o_ref[...] = _bn(v, g_ref[...], be_ref[...])


_layer = pl.pallas_call(
    _layer_body,
    out_shape=jax.ShapeDtypeStruct((_N, _D), jnp.float32),
)


def _leaky(x):
    return jnp.where(x >= 0, x, 0.025 * x)


def _final_body(x_ref, p_ref, Wa_ref, ba_ref, Wb_ref, bb_ref, g_ref, be_ref,
                batch_ref, Wf1_ref, bf1_ref, Wf2_ref, bf2_ref, Wf3_ref,
                bf3_ref, o_ref, emb_ref):
    h = x_ref[...] + p_ref[0] + p_ref[1]
    t = jnp.maximum(
        jnp.dot(h, Wa_ref[...], preferred_element_type=jnp.float32)
        + ba_ref[...], 0.0)
    u = (jnp.dot(t, Wb_ref[...], preferred_element_type=jnp.float32)
         + bb_ref[...])
    v = _bn(jnp.maximum(u, 0.0), g_ref[...], be_ref[...])

    batch = batch_ref[...]
    neg = jnp.float32(-jnp.inf)

    def _seg(g, carry):
        m = jnp.max(jnp.where(batch == g, v, neg), axis=0)
        emb_ref[pl.ds(g, 1), :] = m[None, :]
        return carry

    lax.fori_loop(0, _G, _seg, 0)
    emb = emb_ref[...]
    o = _leaky(jnp.dot(emb, Wf1_ref[...], preferred_element_type=jnp.float32)
               + bf1_ref[...])
    o = _leaky(jnp.dot(o, Wf2_ref[...], preferred_element_type=jnp.float32)
               + bf2_ref[...])
    o_ref[...] = (jnp.dot(o, Wf3_ref[...], preferred_element_type=jnp.float32)
                  + bf3_ref[...])


_final = pl.pallas_call(
    _final_body,
    out_shape=jax.ShapeDtypeStruct((_G, _OUT), jnp.float32),
    scratch_shapes=[pltpu.VMEM((_G, _D), jnp.float32)],
)


def kernel(data_base, edge_index_base, batch_base,
           W1a, b1a, W1b, b1b, g1, be1,
           W2a, b2a, W2b, b2b, g2, be2,
           W3a, b3a, W3b, b3b, g3, be3,
           Wf1, bf1, Wf2, bf2, Wf3, bf3):
    src = edge_index_base[0].reshape(_NW, _NCH, _CH)
    dst = edge_index_base[1].reshape(_NW, _NCH, _CH)
    batch2 = batch_base.reshape(_N, 1)

    x = data_base
    p = _segsum(x, src, dst).reshape(2, _N, _D)
    h = _layer(x, p, W1a, b1a, W1b, b1b, g1, be1)
    p = _segsum(h, src, dst).reshape(2, _N, _D)
    h = _layer(h, p, W2a, b2a, W2b, b2b, g2, be2)
    p = _segsum(h, src, dst).reshape(2, _N, _D)
    return _final(h, p, W3a, b3a, W3b, b3b, g3, be3, batch2,
                  Wf1, bf1, Wf2, bf2, Wf3, bf3)


# speedup vs baseline: 1.1828x; 1.1828x over previous
import functools

import jax
import jax.numpy as jnp
from jax import lax
from jax.experimental import pallas as pl
from jax.experimental.pallas import tpu as pltpu
from jax.experimental.pallas import tpu_sc as plsc

_N = 10000
_E = 320000
_D = 128
_G = 64
_OUT = 8

_NC = 2
_NS = 16
_NW = _NC * _NS

_EPT = _E // _NW
_CH = 40
_NCH = _EPT // _CH
_K = 5
_NG = _NCH // _K
_RPT = _N // _NS
_RCH = 125
_NRC = _RPT // _RCH
_ZCH = 25
_NZC = _RPT // _ZCH


def _segsum_body(x_hbm, src_hbm, dst_hbm, out_hbm,
                 src_v, dst_v, r0, r1, r2, r3, r4, zbuf, acc,
                 s0, s1, s2, s3, s4, zsem):
    rows = [r0, r1, r2, r3, r4]
    sems = [s0, s1, s2, s3, s4]
    c = lax.axis_index("c")
    s = lax.axis_index("s")
    wid = c * _NS + s

    pltpu.sync_copy(src_hbm.at[wid], src_v)
    pltpu.sync_copy(dst_hbm.at[wid], dst_v)
    for b in range(_K):
        pltpu.async_copy(x_hbm.at[src_v.at[b]], rows[b], sems[b])

    zero16 = jnp.zeros((16,), jnp.float32)

    def _z(i, carry):
        for k in range(_D // 16):
            zbuf[i, pl.ds(k * 16, 16)] = zero16
        return carry

    lax.fori_loop(0, _ZCH, _z, 0)

    def _zc(r, carry):
        pltpu.async_copy(zbuf, acc.at[pl.ds(s * _RPT + r * _ZCH, _ZCH)],
                         zsem)
        return carry

    lax.fori_loop(0, _NZC, _zc, 0)
    for _ in range(_NZC):
        pltpu.make_async_copy(zbuf, acc.at[pl.ds(s * _RPT, _ZCH)],
                              zsem).wait()
    plsc.subcore_barrier()

    def _grp(g, carry):
        for b in range(_K):
            j = g * _K + b
            pltpu.make_async_copy(x_hbm.at[src_v.at[j]], rows[b],
                                  sems[b]).wait()
            pltpu.sync_copy(rows[b], acc.at[dst_v.at[j]], add=True)

            @pl.when(g < _NG - 1)
            def _():
                pltpu.async_copy(x_hbm.at[src_v.at[j + _K]], rows[b],
                                 sems[b])
        return carry

    lax.fori_loop(0, _NG, _grp, 0)
    plsc.subcore_barrier()

    for r in range(_NRC):
        off = s * _RPT + r * _RCH
        pltpu.async_copy(acc.at[pl.ds(off, _RCH)],
                         out_hbm.at[pl.ds(c * _N + off, _RCH)], zsem)
    for r in range(_NRC):
        off = s * _RPT + r * _RCH
        pltpu.make_async_copy(acc.at[pl.ds(off, _RCH)],
                              out_hbm.at[pl.ds(c * _N + off, _RCH)],
                              zsem).wait()


_segsum = functools.partial(
    pl.kernel,
    out_type=jax.ShapeDtypeStruct((2 * _N, _D), jnp.float32),
    mesh=plsc.VectorSubcoreMesh(core_axis_name="c", subcore_axis_name="s"),
    scratch_types=(
        [pltpu.VMEM((_NCH, _CH), jnp.int32),
         pltpu.VMEM((_NCH, _CH), jnp.int32)]
        + [pltpu.VMEM((_CH, _D), jnp.float32) for _ in range(_K)]
        + [pltpu.VMEM((_ZCH, _D), jnp.float32),
           pltpu.VMEM_SHARED((_N, _D), jnp.float32)]
        + [pltpu.SemaphoreType.DMA for _ in range(_K + 1)]
    ),
    compiler_params=pltpu.CompilerParams(use_tc_tiling_on_sc=False),
)(_segsum_body)


def _bn(v, g, be):
    m = jnp.mean(v, axis=0)
    var = jnp.mean((v - m) * (v - m), axis=0)
    return (v - m) / jnp.sqrt(var + 1e-5) * g + be


def _layer_body(x_ref, p_ref, Wa_ref, ba_ref, Wb_ref, bb_ref, g_ref, be_ref,
                o_ref):
    h = x_ref[...] + p_ref[0] + p_ref[1]
    t = jnp.maximum(
        jnp.dot(h, Wa_ref[...], preferred_element_type=jnp.float32)
        + ba_ref[...], 0.0)
    u = (jnp.dot(t, Wb_ref[...], preferred_element_type=jnp.float32)
         + bb_ref[...])
    v = jnp.maximum(u, 0.0)
    o_ref[...] = _bn(v, g_ref[...], be_ref[...])


_layer = pl.pallas_call(
    _layer_body,
    out_shape=jax.ShapeDtypeStruct((_N, _D), jnp.float32),
)


def _leaky(x):
    return jnp.where(x >= 0, x, 0.025 * x)


def _final_body(x_ref, p_ref, Wa_ref, ba_ref, Wb_ref, bb_ref, g_ref, be_ref,
                batch_ref, Wf1_ref, bf1_ref, Wf2_ref, bf2_ref, Wf3_ref,
                bf3_ref, o_ref, emb_ref):
    h = x_ref[...] + p_ref[0] + p_ref[1]
    t = jnp.maximum(
        jnp.dot(h, Wa_ref[...], preferred_element_type=jnp.float32)
        + ba_ref[...], 0.0)
    u = (jnp.dot(t, Wb_ref[...], preferred_element_type=jnp.float32)
         + bb_ref[...])
    v = _bn(jnp.maximum(u, 0.0), g_ref[...], be_ref[...])

    batch = batch_ref[...]
    neg = jnp.float32(-jnp.inf)

    def _seg(g, carry):
        m = jnp.max(jnp.where(batch == g, v, neg), axis=0)
        emb_ref[pl.ds(g, 1), :] = m[None, :]
        return carry

    lax.fori_loop(0, _G, _seg, 0)
    emb = emb_ref[...]
    o = _leaky(jnp.dot(emb, Wf1_ref[...], preferred_element_type=jnp.float32)
               + bf1_ref[...])
    o = _leaky(jnp.dot(o, Wf2_ref[...], preferred_element_type=jnp.float32)
               + bf2_ref[...])
    o_ref[...] = (jnp.dot(o, Wf3_ref[...], preferred_element_type=jnp.float32)
                  + bf3_ref[...])


_final = pl.pallas_call(
    _final_body,
    out_shape=jax.ShapeDtypeStruct((_G, _OUT), jnp.float32),
    scratch_shapes=[pltpu.VMEM((_G, _D), jnp.float32)],
)


def kernel(data_base, edge_index_base, batch_base,
           W1a, b1a, W1b, b1b, g1, be1,
           W2a, b2a, W2b, b2b, g2, be2,
           W3a, b3a, W3b, b3b, g3, be3,
           Wf1, bf1, Wf2, bf2, Wf3, bf3):
    src = edge_index_base[0].reshape(_NW, _NCH, _CH)
    dst = edge_index_base[1].reshape(_NW, _NCH, _CH)
    batch2 = batch_base.reshape(_N, 1)

    x = data_base
    p = _segsum(x, src, dst).reshape(2, _N, _D)
    h = _layer(x, p, W1a, b1a, W1b, b1b, g1, be1)
    p = _segsum(h, src, dst).reshape(2, _N, _D)
    h = _layer(h, p, W2a, b2a, W2b, b2b, g2, be2)
    p = _segsum(h, src, dst).reshape(2, _N, _D)
    return _final(h, p, W3a, b3a, W3b, b3b, g3, be3, batch2,
                  Wf1, bf1, Wf2, bf2, Wf3, bf3)
